# baseline (speedup 1.0000x reference)
import jax
import jax.numpy as jnp
from jax import lax
from jax.experimental import pallas as pl
from jax.experimental.pallas import tpu as pltpu

B = 8
N = 8192
N_GROUPS = 512
GROUP_SIZE = 32
EMBED_DIM = 384


def _fps_body(xyz_ref, f0_ref, cx_ref, cy_ref, cz_ref):
    x = xyz_ref[0]
    y = xyz_ref[1]
    z = xyz_ref[2]
    col = lax.broadcasted_iota(jnp.int32, (B, N), 1)
    colM = lax.broadcasted_iota(jnp.int32, (B, N_GROUPS), 1)

    def body(i, carry):
        dist, far, ax, ay, az = carry
        onehot = (col == far).astype(jnp.float32)
        cx = jnp.sum(x * onehot, axis=1, keepdims=True)
        cy = jnp.sum(y * onehot, axis=1, keepdims=True)
        cz = jnp.sum(z * onehot, axis=1, keepdims=True)
        hit = colM == i
        ax = jnp.where(hit, cx, ax)
        ay = jnp.where(hit, cy, ay)
        az = jnp.where(hit, cz, az)
        dx = x - cx
        dy = y - cy
        dz = z - cz
        d = dx * dx + dy * dy
        d = d + dz * dz
        dist = jnp.where(d < dist, d, dist)
        m = jnp.max(dist, axis=1, keepdims=True)
        sel = jnp.where(dist == m, col, jnp.int32(N))
        far = jnp.min(sel, axis=1, keepdims=True)
        return dist, far, ax, ay, az

    dist0 = jnp.full((B, N), 1e10, dtype=jnp.float32)
    far0 = f0_ref[...]
    zM = jnp.zeros((B, N_GROUPS), dtype=jnp.float32)
    _, _, ax, ay, az = lax.fori_loop(0, N_GROUPS, body,
                                     (dist0, far0, zM, zM, zM))
    cx_ref[...] = ax
    cy_ref[...] = ay
    cz_ref[...] = az


def _fps_centroids(xyz_t, f0):
    cx, cy, cz = pl.pallas_call(
        _fps_body,
        out_shape=[jax.ShapeDtypeStruct((B, N_GROUPS), jnp.float32)] * 3,
    )(xyz_t, f0)
    return jnp.stack([cx, cy, cz], axis=-1)


_ROWS = B * N_GROUPS * GROUP_SIZE
_RBLK = 2048
_NBLK = _ROWS // _RBLK
_GBLK = _RBLK // GROUP_SIZE


def _mlp_body(x_ref, W1_ref, b1_ref, g1_ref, be1_ref, W2_ref, b2_ref, g2_ref,
              be2_ref, W3_ref, b3_ref, g3_ref, be3_ref, tok_ref,
              s1, q1, s2, q2, s3, q3, gmax, gmin):
    p = pl.program_id(0)
    j = pl.program_id(1)
    eps = jnp.float32(1e-5)
    ntot = jnp.float32(_ROWS)

    def mm(a, w_ref, b_ref):
        y = lax.dot_general(a, w_ref[...], (((1,), (1,)), ((), ())),
                            preferred_element_type=jnp.float32)
        return y + b_ref[...]

    def bn_relu(y, s_ref, q_ref, g_ref, be_ref):
        m = s_ref[...] / ntot
        var = q_ref[...] / ntot - m * m
        inv = lax.rsqrt(var + eps)
        return jnp.maximum(g_ref[...] * (y - m) * inv + be_ref[...], 0.0)

    @pl.when(jnp.logical_and(p == 0, j == 0))
    def _init():
        s1[...] = jnp.zeros_like(s1)
        q1[...] = jnp.zeros_like(q1)
        s2[...] = jnp.zeros_like(s2)
        q2[...] = jnp.zeros_like(q2)
        s3[...] = jnp.zeros_like(s3)
        q3[...] = jnp.zeros_like(q3)

    @pl.when(p == 0)
    def _p0():
        y1 = mm(x_ref[...], W1_ref, b1_ref)
        s1[...] += jnp.sum(y1, axis=0, keepdims=True)
        q1[...] += jnp.sum(y1 * y1, axis=0, keepdims=True)

    @pl.when(p == 1)
    def _p1():
        y1 = mm(x_ref[...], W1_ref, b1_ref)
        h1 = bn_relu(y1, s1, q1, g1_ref, be1_ref)
        y2 = mm(h1, W2_ref, b2_ref)
        s2[...] += jnp.sum(y2, axis=0, keepdims=True)
        q2[...] += jnp.sum(y2 * y2, axis=0, keepdims=True)

    @pl.when(p == 2)
    def _p2():
        y1 = mm(x_ref[...], W1_ref, b1_ref)
        h1 = bn_relu(y1, s1, q1, g1_ref, be1_ref)
        y2 = mm(h1, W2_ref, b2_ref)
        h2 = bn_relu(y2, s2, q2, g2_ref, be2_ref)
        y3 = mm(h2, W3_ref, b3_ref)
        s3[...] += jnp.sum(y3, axis=0, keepdims=True)
        q3[...] += jnp.sum(y3 * y3, axis=0, keepdims=True)
        y3g = y3.reshape(_GBLK, GROUP_SIZE, EMBED_DIM)
        r0 = pl.multiple_of(j * _GBLK, _GBLK)
        gmax[pl.ds(r0, _GBLK), :] = jnp.max(y3g, axis=1)
        gmin[pl.ds(r0, _GBLK), :] = jnp.min(y3g, axis=1)

    @pl.when(p == 3)
    def _p3():
        m3 = s3[...] / ntot
        var3 = q3[...] / ntot - m3 * m3
        inv3 = lax.rsqrt(var3 + eps)
        r0 = pl.multiple_of(j * _GBLK, _GBLK)
        gx = gmax[pl.ds(r0, _GBLK), :]
        gm = gmin[pl.ds(r0, _GBLK), :]
        g3v = g3_ref[...]
        hi = g3v * (gx - m3) * inv3
        lo = g3v * (gm - m3) * inv3
        tok_ref[...] = jnp.where(g3v > 0, hi, lo) + be3_ref[...]


def _mlp_tokens(xrows, W1, b1, g1, be1, W2, b2, g2, be2, W3, b3, g3, be3):
    r2 = lambda a: a.reshape(1, -1)
    out = pl.pallas_call(
        _mlp_body,
        grid=(4, _NBLK),
        in_specs=[
            pl.BlockSpec((_RBLK, 3), lambda p, j: (j, 0)),
            pl.BlockSpec((64, 3), lambda p, j: (0, 0)),
            pl.BlockSpec((1, 64), lambda p, j: (0, 0)),
            pl.BlockSpec((1, 64), lambda p, j: (0, 0)),
            pl.BlockSpec((1, 64), lambda p, j: (0, 0)),
            pl.BlockSpec((128, 64), lambda p, j: (0, 0)),
            pl.BlockSpec((1, 128), lambda p, j: (0, 0)),
            pl.BlockSpec((1, 128), lambda p, j: (0, 0)),
            pl.BlockSpec((1, 128), lambda p, j: (0, 0)),
            pl.BlockSpec((EMBED_DIM, 128), lambda p, j: (0, 0)),
            pl.BlockSpec((1, EMBED_DIM), lambda p, j: (0, 0)),
            pl.BlockSpec((1, EMBED_DIM), lambda p, j: (0, 0)),
            pl.BlockSpec((1, EMBED_DIM), lambda p, j: (0, 0)),
        ],
        out_specs=pl.BlockSpec((_GBLK, EMBED_DIM), lambda p, j: (j, 0)),
        out_shape=jax.ShapeDtypeStruct((B * N_GROUPS, EMBED_DIM), jnp.float32),
        scratch_shapes=[
            pltpu.VMEM((1, 64), jnp.float32), pltpu.VMEM((1, 64), jnp.float32),
            pltpu.VMEM((1, 128), jnp.float32), pltpu.VMEM((1, 128), jnp.float32),
            pltpu.VMEM((1, EMBED_DIM), jnp.float32),
            pltpu.VMEM((1, EMBED_DIM), jnp.float32),
            pltpu.VMEM((B * N_GROUPS, EMBED_DIM), jnp.float32),
            pltpu.VMEM((B * N_GROUPS, EMBED_DIM), jnp.float32),
        ],
        compiler_params=pltpu.CompilerParams(
            dimension_semantics=("arbitrary", "arbitrary")),
    )(xrows, W1, r2(b1), r2(g1), r2(be1), W2, r2(b2), r2(g2), r2(be2),
      W3, r2(b3), r2(g3), r2(be3))
    return out


def kernel(points_data, W1, b1, g1, be1, W2, b2, g2, be2, W3, b3, g3, be3):
    xyz = points_data
    xyz_t = jnp.transpose(xyz, (2, 0, 1))
    f0 = jax.random.randint(jax.random.key(42), (B,), 0, N,
                            dtype=jnp.int32).reshape(B, 1)
    centroids_xyz = _fps_centroids(xyz_t, f0)

    d2 = (jnp.sum(centroids_xyz ** 2, axis=-1)[:, :, None]
          + jnp.sum(xyz ** 2, axis=-1)[:, None, :]
          - 2.0 * jnp.einsum('bmc,bnc->bmn', centroids_xyz, xyz))
    _, idx = lax.top_k(-d2, GROUP_SIZE)
    grouped = jnp.take_along_axis(
        xyz, idx.reshape(B, N_GROUPS * GROUP_SIZE)[:, :, None], axis=1)
    grouped = grouped.reshape(B, N_GROUPS, GROUP_SIZE, 3)
    gn = grouped - centroids_xyz[:, :, None, :]

    xrows = gn.reshape(_ROWS, 3)
    tok = _mlp_tokens(xrows, W1, b1, g1, be1, W2, b2, g2, be2, W3, b3, g3, be3)
    tokens = tok.reshape(B, N_GROUPS, EMBED_DIM)
    return (tokens, centroids_xyz)


# speedup vs baseline: 6.1436x; 6.1436x over previous
import functools

import jax
import jax.numpy as jnp
from jax import lax
from jax.experimental import pallas as pl
from jax.experimental.pallas import tpu as pltpu
from jax.experimental.pallas import tpu_sc as plsc

B = 8
N = 8192
N_GROUPS = 512
GROUP_SIZE = 32
EMBED_DIM = 384


def _fps_body(xyz_ref, f0_ref, cx_ref, cy_ref, cz_ref):
    x = xyz_ref[:, 0, :]
    y = xyz_ref[:, 1, :]
    z = xyz_ref[:, 2, :]
    col = lax.broadcasted_iota(jnp.int32, (B, N), 1)
    colM = lax.broadcasted_iota(jnp.int32, (B, N_GROUPS), 1)

    def body(i, carry):
        dist, far, ax, ay, az = carry
        onehot = (col == far).astype(jnp.float32)
        cx = jnp.sum(x * onehot, axis=1, keepdims=True)
        cy = jnp.sum(y * onehot, axis=1, keepdims=True)
        cz = jnp.sum(z * onehot, axis=1, keepdims=True)
        hit = colM == i
        ax = jnp.where(hit, cx, ax)
        ay = jnp.where(hit, cy, ay)
        az = jnp.where(hit, cz, az)
        dx = x - cx
        dy = y - cy
        dz = z - cz
        d = dx * dx + dy * dy
        d = d + dz * dz
        dist = jnp.where(d < dist, d, dist)
        m = jnp.max(dist, axis=1, keepdims=True)
        sel = jnp.where(dist == m, col, jnp.int32(N))
        far = jnp.min(sel, axis=1, keepdims=True)
        return dist, far, ax, ay, az

    dist0 = jnp.full((B, N), 1e10, dtype=jnp.float32)
    far0 = f0_ref[...]
    zM = jnp.zeros((B, N_GROUPS), dtype=jnp.float32)
    _, _, ax, ay, az = lax.fori_loop(0, N_GROUPS, body,
                                     (dist0, far0, zM, zM, zM))
    cx_ref[...] = ax
    cy_ref[...] = ay
    cz_ref[...] = az


def _fps_centroids(xyz_t, f0):
    cx, cy, cz = pl.pallas_call(
        _fps_body,
        out_shape=[jax.ShapeDtypeStruct((B, N_GROUPS), jnp.float32)] * 3,
    )(xyz_t, f0)
    return jnp.stack([cx, cy, cz], axis=-1)


_ROWS = B * N_GROUPS * GROUP_SIZE


_MBLK = 256
_BISECT = 26


def _d2thr_body(cen_ref, xyz_ref, d2_ref, thr_ref):
    cen = cen_ref[0]
    P = xyz_ref[0]
    G = lax.dot_general(cen, P, (((1,), (0,)), ((), ())),
                        preferred_element_type=jnp.float32)
    cn2 = jnp.sum(P * P, axis=0, keepdims=True)
    cm2 = jnp.sum(cen * cen, axis=1, keepdims=True)
    d2 = (cm2 + cn2) - 2.0 * G
    d2_ref[0] = d2

    cmin = jnp.min(d2.reshape(_MBLK, GROUP_SIZE, N // GROUP_SIZE), axis=2)
    hi0 = jnp.max(cmin, axis=1, keepdims=True)
    lo0 = jnp.min(cmin, axis=1, keepdims=True) - 1.0
    kf = jnp.float32(GROUP_SIZE)

    def bis(_, carry):
        lo, hi, t, found = carry
        mid = 0.5 * (lo + hi)
        cnt = jnp.sum((d2 <= mid).astype(jnp.float32), axis=1, keepdims=True)
        live = found == 0.0
        eq = jnp.logical_and(cnt == kf, live)
        t = jnp.where(eq, mid, t)
        found = jnp.where(eq, 1.0, found)
        ge = cnt >= kf
        live2 = jnp.logical_and(live, jnp.logical_not(eq))
        hi = jnp.where(jnp.logical_and(live2, ge), mid, hi)
        lo = jnp.where(jnp.logical_and(live2, jnp.logical_not(ge)), mid, lo)
        return lo, hi, t, found

    f0 = jnp.zeros((_MBLK, 1), dtype=jnp.float32)
    lo, hi, t, found = lax.fori_loop(0, _BISECT, bis, (lo0, hi0, hi0, f0))
    thr_ref[0] = jnp.where(found > 0.0, t, hi)


def _d2_thresholds(cen, xyz_t):
    d2, thr = pl.pallas_call(
        _d2thr_body,
        grid=(B, N_GROUPS // _MBLK),
        in_specs=[
            pl.BlockSpec((1, _MBLK, 3), lambda b, r: (b, r, 0)),
            pl.BlockSpec((1, 3, N), lambda b, r: (b, 0, 0)),
        ],
        out_specs=[
            pl.BlockSpec((1, _MBLK, N), lambda b, r: (b, r, 0)),
            pl.BlockSpec((1, _MBLK, 1), lambda b, r: (b, r, 0)),
        ],
        out_shape=[
            jax.ShapeDtypeStruct((B, N_GROUPS, N), jnp.float32),
            jax.ShapeDtypeStruct((B, N_GROUPS, 1), jnp.float32),
        ],
        compiler_params=pltpu.CompilerParams(
            dimension_semantics=("arbitrary", "arbitrary")),
    )(cen, xyz_t)
    return d2.reshape(B * N_GROUPS, N), thr.reshape(B * N_GROUPS)


_NW = 32
_NROWS = B * N_GROUPS
_RPT = _NROWS // _NW
_PPT = _RPT * GROUP_SIZE
_NV = N // 16


def _sc_body(d2_hbm, thr_hbm, cen_hbm, pts_hbm, out_hbm,
             d2a, d2b, thr_v, cxv, cyv, czv, xp, yp, zp,
             bufx, bufy, bufz, selbuf, sema, semb):
    cc = lax.axis_index("c")
    ss = lax.axis_index("s")
    wid = ss * 2 + cc
    base_row = wid * _RPT
    bt = wid // (_NW // B)

    pltpu.sync_copy(thr_hbm.at[pl.ds(base_row, _RPT)], thr_v)
    pltpu.sync_copy(cen_hbm.at[pl.ds(base_row, _RPT)], cxv)
    pltpu.sync_copy(cen_hbm.at[pl.ds(_NROWS + base_row, _RPT)], cyv)
    pltpu.sync_copy(cen_hbm.at[pl.ds(2 * _NROWS + base_row, _RPT)], czv)
    pb = bt * 3 * N
    pltpu.sync_copy(pts_hbm.at[pl.ds(pb, N)], xp)
    pltpu.sync_copy(pts_hbm.at[pl.ds(pb + N, N)], yp)
    pltpu.sync_copy(pts_hbm.at[pl.ds(pb + 2 * N, N)], zp)

    iota16 = lax.iota(jnp.int32, 16)

    def process_row(r, d2row):
        rsplat = jnp.broadcast_to(r, (16,)).astype(jnp.int32)
        tv = plsc.load_gather(thr_v, [rsplat])

        def scan_body(ci, p):
            v = d2row[pl.ds(pl.multiple_of(ci * 16, 16), 16)]
            m = v <= tv
            idx = iota16 + ci * 16
            plsc.store_compressed(selbuf.at[pl.ds(p, 16)], idx, mask=m)
            return p + jnp.sum(m.astype(jnp.int32))

        lax.fori_loop(0, _NV, scan_body, jnp.int32(0))

        i0 = selbuf[pl.ds(0, 16)]
        i1 = selbuf[pl.ds(16, 16)]
        cx = plsc.load_gather(cxv, [rsplat])
        cy = plsc.load_gather(cyv, [rsplat])
        cz = plsc.load_gather(czv, [rsplat])
        ob = pl.multiple_of(r * GROUP_SIZE, GROUP_SIZE)
        bufx[pl.ds(ob, 16)] = plsc.load_gather(xp, [i0]) - cx
        bufx[pl.ds(ob + 16, 16)] = plsc.load_gather(xp, [i1]) - cx
        bufy[pl.ds(ob, 16)] = plsc.load_gather(yp, [i0]) - cy
        bufy[pl.ds(ob + 16, 16)] = plsc.load_gather(yp, [i1]) - cy
        bufz[pl.ds(ob, 16)] = plsc.load_gather(zp, [i0]) - cz
        bufz[pl.ds(ob + 16, 16)] = plsc.load_gather(zp, [i1]) - cz

    def rowslice(r):
        return d2_hbm.at[pl.ds(pl.multiple_of((base_row + r) * N, N), N)]

    pltpu.async_copy(rowslice(0), d2a, sema)

    def two_rows(i, _):
        ra = 2 * i
        pltpu.async_copy(rowslice(ra + 1), d2b, semb)
        pltpu.make_async_copy(rowslice(0), d2a, sema).wait()
        process_row(ra, d2a)

        @pl.when(ra + 2 < _RPT)
        def _():
            pltpu.async_copy(rowslice(ra + 2), d2a, sema)

        pltpu.make_async_copy(rowslice(0), d2b, semb).wait()
        process_row(ra + 1, d2b)
        return 0

    lax.fori_loop(0, _RPT // 2, two_rows, 0)

    ob0 = wid * _PPT
    pltpu.sync_copy(bufx, out_hbm.at[pl.ds(ob0, _PPT)])
    pltpu.sync_copy(bufy, out_hbm.at[pl.ds(_ROWS + ob0, _PPT)])
    pltpu.sync_copy(bufz, out_hbm.at[pl.ds(2 * _ROWS + ob0, _PPT)])


_sc_select_gather = functools.partial(
    pl.kernel,
    out_type=jax.ShapeDtypeStruct((3 * _ROWS,), jnp.float32),
    mesh=plsc.VectorSubcoreMesh(core_axis_name="c", subcore_axis_name="s"),
    compiler_params=pltpu.CompilerParams(needs_layout_passes=False),
    scratch_types=[
        pltpu.VMEM((N,), jnp.float32),
        pltpu.VMEM((N,), jnp.float32),
        pltpu.VMEM((_RPT,), jnp.float32),
        pltpu.VMEM((_RPT,), jnp.float32),
        pltpu.VMEM((_RPT,), jnp.float32),
        pltpu.VMEM((_RPT,), jnp.float32),
        pltpu.VMEM((N,), jnp.float32),
        pltpu.VMEM((N,), jnp.float32),
        pltpu.VMEM((N,), jnp.float32),
        pltpu.VMEM((_PPT,), jnp.float32),
        pltpu.VMEM((_PPT,), jnp.float32),
        pltpu.VMEM((_PPT,), jnp.float32),
        pltpu.VMEM((N,), jnp.int32),
        pltpu.SemaphoreType.DMA,
        pltpu.SemaphoreType.DMA,
    ],
)(_sc_body)


_RBLK = 2048
_NBLK = _ROWS // _RBLK
_GBLK = _RBLK // GROUP_SIZE


def _mlp_body(x_ref, W1_ref, b1_ref, g1_ref, be1_ref, W2_ref, b2_ref, g2_ref,
              be2_ref, W3_ref, b3_ref, g3_ref, be3_ref, tok_ref,
              s1, q1, s2, q2, s3, q3, gmax, gmin):
    p = pl.program_id(0)
    j = pl.program_id(1)
    eps = jnp.float32(1e-5)
    ntot = jnp.float32(_ROWS)

    def mm(a, w_ref, b_ref):
        y = lax.dot_general(a, w_ref[...], (((1,), (1,)), ((), ())),
                            preferred_element_type=jnp.float32)
        return y + b_ref[...]

    def bn_relu(y, s_ref, q_ref, g_ref, be_ref):
        m = s_ref[...] / ntot
        var = q_ref[...] / ntot - m * m
        inv = lax.rsqrt(var + eps)
        return jnp.maximum(g_ref[...] * (y - m) * inv + be_ref[...], 0.0)

    @pl.when(jnp.logical_and(p == 0, j == 0))
    def _init():
        s1[...] = jnp.zeros_like(s1)
        q1[...] = jnp.zeros_like(q1)
        s2[...] = jnp.zeros_like(s2)
        q2[...] = jnp.zeros_like(q2)
        s3[...] = jnp.zeros_like(s3)
        q3[...] = jnp.zeros_like(q3)

    @pl.when(p == 0)
    def _p0():
        y1 = mm(x_ref[...], W1_ref, b1_ref)
        s1[...] += jnp.sum(y1, axis=0, keepdims=True)
        q1[...] += jnp.sum(y1 * y1, axis=0, keepdims=True)

    @pl.when(p == 1)
    def _p1():
        y1 = mm(x_ref[...], W1_ref, b1_ref)
        h1 = bn_relu(y1, s1, q1, g1_ref, be1_ref)
        y2 = mm(h1, W2_ref, b2_ref)
        s2[...] += jnp.sum(y2, axis=0, keepdims=True)
        q2[...] += jnp.sum(y2 * y2, axis=0, keepdims=True)

    @pl.when(p == 2)
    def _p2():
        y1 = mm(x_ref[...], W1_ref, b1_ref)
        h1 = bn_relu(y1, s1, q1, g1_ref, be1_ref)
        y2 = mm(h1, W2_ref, b2_ref)
        h2 = bn_relu(y2, s2, q2, g2_ref, be2_ref)
        y3 = mm(h2, W3_ref, b3_ref)
        s3[...] += jnp.sum(y3, axis=0, keepdims=True)
        q3[...] += jnp.sum(y3 * y3, axis=0, keepdims=True)
        y3g = y3.reshape(_GBLK, GROUP_SIZE, EMBED_DIM)
        r0 = pl.multiple_of(j * _GBLK, _GBLK)
        gmax[pl.ds(r0, _GBLK), :] = jnp.max(y3g, axis=1)
        gmin[pl.ds(r0, _GBLK), :] = jnp.min(y3g, axis=1)

    @pl.when(p == 3)
    def _p3():
        m3 = s3[...] / ntot
        var3 = q3[...] / ntot - m3 * m3
        inv3 = lax.rsqrt(var3 + eps)
        r0 = pl.multiple_of(j * _GBLK, _GBLK)
        gx = gmax[pl.ds(r0, _GBLK), :]
        gm = gmin[pl.ds(r0, _GBLK), :]
        g3v = g3_ref[...]
        hi = g3v * (gx - m3) * inv3
        lo = g3v * (gm - m3) * inv3
        tok_ref[...] = jnp.where(g3v > 0, hi, lo) + be3_ref[...]


def _mlp_tokens(xrows, W1, b1, g1, be1, W2, b2, g2, be2, W3, b3, g3, be3):
    r2 = lambda a: a.reshape(1, -1)
    out = pl.pallas_call(
        _mlp_body,
        grid=(4, _NBLK),
        in_specs=[
            pl.BlockSpec((_RBLK, 3), lambda p, j: (j, 0)),
            pl.BlockSpec((64, 3), lambda p, j: (0, 0)),
            pl.BlockSpec((1, 64), lambda p, j: (0, 0)),
            pl.BlockSpec((1, 64), lambda p, j: (0, 0)),
            pl.BlockSpec((1, 64), lambda p, j: (0, 0)),
            pl.BlockSpec((128, 64), lambda p, j: (0, 0)),
            pl.BlockSpec((1, 128), lambda p, j: (0, 0)),
            pl.BlockSpec((1, 128), lambda p, j: (0, 0)),
            pl.BlockSpec((1, 128), lambda p, j: (0, 0)),
            pl.BlockSpec((EMBED_DIM, 128), lambda p, j: (0, 0)),
            pl.BlockSpec((1, EMBED_DIM), lambda p, j: (0, 0)),
            pl.BlockSpec((1, EMBED_DIM), lambda p, j: (0, 0)),
            pl.BlockSpec((1, EMBED_DIM), lambda p, j: (0, 0)),
        ],
        out_specs=pl.BlockSpec((_GBLK, EMBED_DIM), lambda p, j: (j, 0)),
        out_shape=jax.ShapeDtypeStruct((B * N_GROUPS, EMBED_DIM), jnp.float32),
        scratch_shapes=[
            pltpu.VMEM((1, 64), jnp.float32), pltpu.VMEM((1, 64), jnp.float32),
            pltpu.VMEM((1, 128), jnp.float32), pltpu.VMEM((1, 128), jnp.float32),
            pltpu.VMEM((1, EMBED_DIM), jnp.float32),
            pltpu.VMEM((1, EMBED_DIM), jnp.float32),
            pltpu.VMEM((B * N_GROUPS, EMBED_DIM), jnp.float32),
            pltpu.VMEM((B * N_GROUPS, EMBED_DIM), jnp.float32),
        ],
        compiler_params=pltpu.CompilerParams(
            dimension_semantics=("arbitrary", "arbitrary")),
    )(xrows, W1, r2(b1), r2(g1), r2(be1), W2, r2(b2), r2(g2), r2(be2),
      W3, r2(b3), r2(g3), r2(be3))
    return out


def kernel(points_data, W1, b1, g1, be1, W2, b2, g2, be2, W3, b3, g3, be3):
    xyz = points_data
    xyz_t = jnp.transpose(xyz, (0, 2, 1))
    f0 = jax.random.randint(jax.random.key(42), (B,), 0, N,
                            dtype=jnp.int32).reshape(B, 1)
    centroids_xyz = _fps_centroids(xyz_t, f0)

    d2, thr = _d2_thresholds(centroids_xyz, xyz_t)
    cen_flat = jnp.transpose(centroids_xyz.reshape(_NROWS, 3)).reshape(-1)
    gn_flat = _sc_select_gather(d2.reshape(-1), thr, cen_flat,
                                xyz_t.reshape(-1))
    xrows = jnp.transpose(gn_flat.reshape(3, _ROWS))
    tok = _mlp_tokens(xrows, W1, b1, g1, be1, W2, b2, g2, be2, W3, b3, g3, be3)
    tokens = tok.reshape(B, N_GROUPS, EMBED_DIM)
    return (tokens, centroids_xyz)


# speedup vs baseline: 7.2608x; 1.1818x over previous
import functools

import jax
import jax.numpy as jnp
from jax import lax
from jax.experimental import pallas as pl
from jax.experimental.pallas import tpu as pltpu
from jax.experimental.pallas import tpu_sc as plsc

B = 8
N = 8192
N_GROUPS = 512
GROUP_SIZE = 32
EMBED_DIM = 384


def _fps_body(xyz_ref, f0_ref, cx_ref, cy_ref, cz_ref):
    x = xyz_ref[:, 0, :]
    y = xyz_ref[:, 1, :]
    z = xyz_ref[:, 2, :]
    col = lax.broadcasted_iota(jnp.int32, (B, N), 1)
    colM = lax.broadcasted_iota(jnp.int32, (B, N_GROUPS), 1)

    def body(i, carry):
        dist, far, ax, ay, az = carry
        onehot = (col == far).astype(jnp.float32)
        cx = jnp.sum(x * onehot, axis=1, keepdims=True)
        cy = jnp.sum(y * onehot, axis=1, keepdims=True)
        cz = jnp.sum(z * onehot, axis=1, keepdims=True)
        hit = colM == i
        ax = jnp.where(hit, cx, ax)
        ay = jnp.where(hit, cy, ay)
        az = jnp.where(hit, cz, az)
        dx = x - cx
        dy = y - cy
        dz = z - cz
        d = dx * dx + dy * dy
        d = d + dz * dz
        dist = jnp.where(d < dist, d, dist)
        m = jnp.max(dist, axis=1, keepdims=True)
        sel = jnp.where(dist == m, col, jnp.int32(N))
        far = jnp.min(sel, axis=1, keepdims=True)
        return dist, far, ax, ay, az

    dist0 = jnp.full((B, N), 1e10, dtype=jnp.float32)
    far0 = f0_ref[...]
    zM = jnp.zeros((B, N_GROUPS), dtype=jnp.float32)
    _, _, ax, ay, az = lax.fori_loop(0, N_GROUPS, body,
                                     (dist0, far0, zM, zM, zM))
    cx_ref[...] = ax
    cy_ref[...] = ay
    cz_ref[...] = az


def _fps_centroids(xyz_t, f0):
    cx, cy, cz = pl.pallas_call(
        _fps_body,
        out_shape=[jax.ShapeDtypeStruct((B, N_GROUPS), jnp.float32)] * 3,
    )(xyz_t, f0)
    return jnp.stack([cx, cy, cz], axis=-1)


_ROWS = B * N_GROUPS * GROUP_SIZE


_MBLK = 256
_BISECT = 26


def _d2thr_body(cen_ref, xyz_ref, d2_ref, thr_ref):
    cen = cen_ref[0]
    P = xyz_ref[0]
    G = lax.dot_general(cen, P, (((1,), (0,)), ((), ())),
                        preferred_element_type=jnp.float32)
    cn2 = jnp.sum(P * P, axis=0, keepdims=True)
    cm2 = jnp.sum(cen * cen, axis=1, keepdims=True)
    d2 = (cm2 + cn2) - 2.0 * G
    d2_ref[0] = d2

    cmin = jnp.min(d2.reshape(_MBLK, GROUP_SIZE, N // GROUP_SIZE), axis=2)
    hi0 = jnp.max(cmin, axis=1, keepdims=True)
    lo0 = jnp.min(cmin, axis=1, keepdims=True) - 1.0
    kf = jnp.float32(GROUP_SIZE)

    def bis(_, carry):
        lo, hi, t, found = carry
        mid = 0.5 * (lo + hi)
        cnt = jnp.sum((d2 <= mid).astype(jnp.float32), axis=1, keepdims=True)
        live = found == 0.0
        eq = jnp.logical_and(cnt == kf, live)
        t = jnp.where(eq, mid, t)
        found = jnp.where(eq, 1.0, found)
        ge = cnt >= kf
        live2 = jnp.logical_and(live, jnp.logical_not(eq))
        hi = jnp.where(jnp.logical_and(live2, ge), mid, hi)
        lo = jnp.where(jnp.logical_and(live2, jnp.logical_not(ge)), mid, lo)
        return lo, hi, t, found

    f0 = jnp.zeros((_MBLK, 1), dtype=jnp.float32)
    lo, hi, t, found = lax.fori_loop(0, _BISECT, bis, (lo0, hi0, hi0, f0))
    thr_ref[0] = jnp.where(found > 0.0, t, hi)


def _d2_thresholds(cen, xyz_t):
    d2, thr = pl.pallas_call(
        _d2thr_body,
        grid=(B, N_GROUPS // _MBLK),
        in_specs=[
            pl.BlockSpec((1, _MBLK, 3), lambda b, r: (b, r, 0)),
            pl.BlockSpec((1, 3, N), lambda b, r: (b, 0, 0)),
        ],
        out_specs=[
            pl.BlockSpec((1, _MBLK, N), lambda b, r: (b, r, 0)),
            pl.BlockSpec((1, _MBLK, 1), lambda b, r: (b, r, 0)),
        ],
        out_shape=[
            jax.ShapeDtypeStruct((B, N_GROUPS, N), jnp.float32),
            jax.ShapeDtypeStruct((B, N_GROUPS, 1), jnp.float32),
        ],
        compiler_params=pltpu.CompilerParams(
            dimension_semantics=("arbitrary", "arbitrary")),
    )(cen, xyz_t)
    return d2.reshape(B * N_GROUPS, N), thr.reshape(B * N_GROUPS)


_NW = 32
_NROWS = B * N_GROUPS
_RPT = _NROWS // _NW
_PPT = _RPT * GROUP_SIZE
_NV = N // 16


def _sc_body(d2_hbm, thr_hbm, cen_hbm, pts_hbm, out_hbm,
             d2a, d2b, thr_v, cxv, cyv, czv, xp, yp, zp,
             bufx, bufy, bufz, selbuf, sema, semb):
    cc = lax.axis_index("c")
    ss = lax.axis_index("s")
    wid = ss * 2 + cc
    base_row = wid * _RPT
    bt = wid // (_NW // B)

    pltpu.sync_copy(thr_hbm.at[pl.ds(base_row, _RPT)], thr_v)
    pltpu.sync_copy(cen_hbm.at[pl.ds(base_row, _RPT)], cxv)
    pltpu.sync_copy(cen_hbm.at[pl.ds(_NROWS + base_row, _RPT)], cyv)
    pltpu.sync_copy(cen_hbm.at[pl.ds(2 * _NROWS + base_row, _RPT)], czv)
    pb = bt * 3 * N
    pltpu.sync_copy(pts_hbm.at[pl.ds(pb, N)], xp)
    pltpu.sync_copy(pts_hbm.at[pl.ds(pb + N, N)], yp)
    pltpu.sync_copy(pts_hbm.at[pl.ds(pb + 2 * N, N)], zp)

    iota16 = lax.iota(jnp.int32, 16)

    def process_row(r, d2row):
        rsplat = jnp.broadcast_to(r, (16,)).astype(jnp.int32)
        tv = plsc.load_gather(thr_v, [rsplat])

        def scan_body(ci, p):
            b0 = ci * 64
            vs = [d2row[pl.ds(pl.multiple_of(b0 + k * 16, 16), 16)]
                  for k in range(4)]
            ms = [v <= tv for v in vs]
            pcs = [plsc.all_reduce_population_count(m)[0] for m in ms]
            for k in range(4):
                plsc.store_compressed(selbuf.at[pl.ds(p, 16)],
                                      iota16 + (b0 + k * 16), mask=ms[k])
                p = p + pcs[k]
            return p

        lax.fori_loop(0, _NV // 4, scan_body, jnp.int32(0))

        i0 = selbuf[pl.ds(0, 16)]
        i1 = selbuf[pl.ds(16, 16)]
        cx = plsc.load_gather(cxv, [rsplat])
        cy = plsc.load_gather(cyv, [rsplat])
        cz = plsc.load_gather(czv, [rsplat])
        ob = pl.multiple_of(r * GROUP_SIZE, GROUP_SIZE)
        bufx[pl.ds(ob, 16)] = plsc.load_gather(xp, [i0]) - cx
        bufx[pl.ds(ob + 16, 16)] = plsc.load_gather(xp, [i1]) - cx
        bufy[pl.ds(ob, 16)] = plsc.load_gather(yp, [i0]) - cy
        bufy[pl.ds(ob + 16, 16)] = plsc.load_gather(yp, [i1]) - cy
        bufz[pl.ds(ob, 16)] = plsc.load_gather(zp, [i0]) - cz
        bufz[pl.ds(ob + 16, 16)] = plsc.load_gather(zp, [i1]) - cz

    def rowslice(r):
        return d2_hbm.at[pl.ds(pl.multiple_of((base_row + r) * N, N), N)]

    pltpu.async_copy(rowslice(0), d2a, sema)

    def two_rows(i, _):
        ra = 2 * i
        pltpu.async_copy(rowslice(ra + 1), d2b, semb)
        pltpu.make_async_copy(rowslice(0), d2a, sema).wait()
        process_row(ra, d2a)

        @pl.when(ra + 2 < _RPT)
        def _():
            pltpu.async_copy(rowslice(ra + 2), d2a, sema)

        pltpu.make_async_copy(rowslice(0), d2b, semb).wait()
        process_row(ra + 1, d2b)
        return 0

    lax.fori_loop(0, _RPT // 2, two_rows, 0)

    ob0 = wid * _PPT
    pltpu.sync_copy(bufx, out_hbm.at[pl.ds(ob0, _PPT)])
    pltpu.sync_copy(bufy, out_hbm.at[pl.ds(_ROWS + ob0, _PPT)])
    pltpu.sync_copy(bufz, out_hbm.at[pl.ds(2 * _ROWS + ob0, _PPT)])


_sc_select_gather = functools.partial(
    pl.kernel,
    out_type=jax.ShapeDtypeStruct((3 * _ROWS,), jnp.float32),
    mesh=plsc.VectorSubcoreMesh(core_axis_name="c", subcore_axis_name="s"),
    compiler_params=pltpu.CompilerParams(needs_layout_passes=False),
    scratch_types=[
        pltpu.VMEM((N,), jnp.float32),
        pltpu.VMEM((N,), jnp.float32),
        pltpu.VMEM((_RPT,), jnp.float32),
        pltpu.VMEM((_RPT,), jnp.float32),
        pltpu.VMEM((_RPT,), jnp.float32),
        pltpu.VMEM((_RPT,), jnp.float32),
        pltpu.VMEM((N,), jnp.float32),
        pltpu.VMEM((N,), jnp.float32),
        pltpu.VMEM((N,), jnp.float32),
        pltpu.VMEM((_PPT,), jnp.float32),
        pltpu.VMEM((_PPT,), jnp.float32),
        pltpu.VMEM((_PPT,), jnp.float32),
        pltpu.VMEM((N,), jnp.int32),
        pltpu.SemaphoreType.DMA,
        pltpu.SemaphoreType.DMA,
    ],
)(_sc_body)


_RBLK = 2048
_NBLK = _ROWS // _RBLK
_GBLK = _RBLK // GROUP_SIZE


def _mlp_body(x_ref, W1_ref, b1_ref, g1_ref, be1_ref, W2_ref, b2_ref, g2_ref,
              be2_ref, W3_ref, b3_ref, g3_ref, be3_ref, tok_ref,
              s1, q1, s2, q2, s3, q3, gmax, gmin):
    p = pl.program_id(0)
    j = pl.program_id(1)
    eps = jnp.float32(1e-5)
    ntot = jnp.float32(_ROWS)

    def mm(a, w_ref, b_ref):
        y = lax.dot_general(a, w_ref[...], (((1,), (1,)), ((), ())),
                            preferred_element_type=jnp.float32)
        return y + b_ref[...]

    def bn_relu(y, s_ref, q_ref, g_ref, be_ref):
        m = s_ref[...] / ntot
        var = q_ref[...] / ntot - m * m
        inv = lax.rsqrt(var + eps)
        return jnp.maximum(g_ref[...] * (y - m) * inv + be_ref[...], 0.0)

    @pl.when(jnp.logical_and(p == 0, j == 0))
    def _init():
        s1[...] = jnp.zeros_like(s1)
        q1[...] = jnp.zeros_like(q1)
        s2[...] = jnp.zeros_like(s2)
        q2[...] = jnp.zeros_like(q2)
        s3[...] = jnp.zeros_like(s3)
        q3[...] = jnp.zeros_like(q3)

    @pl.when(p == 0)
    def _p0():
        y1 = mm(x_ref[...], W1_ref, b1_ref)
        s1[...] += jnp.sum(y1, axis=0, keepdims=True)
        q1[...] += jnp.sum(y1 * y1, axis=0, keepdims=True)

    @pl.when(p == 1)
    def _p1():
        y1 = mm(x_ref[...], W1_ref, b1_ref)
        h1 = bn_relu(y1, s1, q1, g1_ref, be1_ref)
        y2 = mm(h1, W2_ref, b2_ref)
        s2[...] += jnp.sum(y2, axis=0, keepdims=True)
        q2[...] += jnp.sum(y2 * y2, axis=0, keepdims=True)

    @pl.when(p == 2)
    def _p2():
        y1 = mm(x_ref[...], W1_ref, b1_ref)
        h1 = bn_relu(y1, s1, q1, g1_ref, be1_ref)
        y2 = mm(h1, W2_ref, b2_ref)
        h2 = bn_relu(y2, s2, q2, g2_ref, be2_ref)
        y3 = mm(h2, W3_ref, b3_ref)
        s3[...] += jnp.sum(y3, axis=0, keepdims=True)
        q3[...] += jnp.sum(y3 * y3, axis=0, keepdims=True)
        y3g = y3.reshape(_GBLK, GROUP_SIZE, EMBED_DIM)
        r0 = pl.multiple_of(j * _GBLK, _GBLK)
        gmax[pl.ds(r0, _GBLK), :] = jnp.max(y3g, axis=1)
        gmin[pl.ds(r0, _GBLK), :] = jnp.min(y3g, axis=1)

    @pl.when(p == 3)
    def _p3():
        m3 = s3[...] / ntot
        var3 = q3[...] / ntot - m3 * m3
        inv3 = lax.rsqrt(var3 + eps)
        r0 = pl.multiple_of(j * _GBLK, _GBLK)
        gx = gmax[pl.ds(r0, _GBLK), :]
        gm = gmin[pl.ds(r0, _GBLK), :]
        g3v = g3_ref[...]
        hi = g3v * (gx - m3) * inv3
        lo = g3v * (gm - m3) * inv3
        tok_ref[...] = jnp.where(g3v > 0, hi, lo) + be3_ref[...]


def _mlp_tokens(xrows, W1, b1, g1, be1, W2, b2, g2, be2, W3, b3, g3, be3):
    r2 = lambda a: a.reshape(1, -1)
    out = pl.pallas_call(
        _mlp_body,
        grid=(4, _NBLK),
        in_specs=[
            pl.BlockSpec((_RBLK, 3), lambda p, j: (j, 0)),
            pl.BlockSpec((64, 3), lambda p, j: (0, 0)),
            pl.BlockSpec((1, 64), lambda p, j: (0, 0)),
            pl.BlockSpec((1, 64), lambda p, j: (0, 0)),
            pl.BlockSpec((1, 64), lambda p, j: (0, 0)),
            pl.BlockSpec((128, 64), lambda p, j: (0, 0)),
            pl.BlockSpec((1, 128), lambda p, j: (0, 0)),
            pl.BlockSpec((1, 128), lambda p, j: (0, 0)),
            pl.BlockSpec((1, 128), lambda p, j: (0, 0)),
            pl.BlockSpec((EMBED_DIM, 128), lambda p, j: (0, 0)),
            pl.BlockSpec((1, EMBED_DIM), lambda p, j: (0, 0)),
            pl.BlockSpec((1, EMBED_DIM), lambda p, j: (0, 0)),
            pl.BlockSpec((1, EMBED_DIM), lambda p, j: (0, 0)),
        ],
        out_specs=pl.BlockSpec((_GBLK, EMBED_DIM), lambda p, j: (j, 0)),
        out_shape=jax.ShapeDtypeStruct((B * N_GROUPS, EMBED_DIM), jnp.float32),
        scratch_shapes=[
            pltpu.VMEM((1, 64), jnp.float32), pltpu.VMEM((1, 64), jnp.float32),
            pltpu.VMEM((1, 128), jnp.float32), pltpu.VMEM((1, 128), jnp.float32),
            pltpu.VMEM((1, EMBED_DIM), jnp.float32),
            pltpu.VMEM((1, EMBED_DIM), jnp.float32),
            pltpu.VMEM((B * N_GROUPS, EMBED_DIM), jnp.float32),
            pltpu.VMEM((B * N_GROUPS, EMBED_DIM), jnp.float32),
        ],
        compiler_params=pltpu.CompilerParams(
            dimension_semantics=("arbitrary", "arbitrary")),
    )(xrows, W1, r2(b1), r2(g1), r2(be1), W2, r2(b2), r2(g2), r2(be2),
      W3, r2(b3), r2(g3), r2(be3))
    return out


def kernel(points_data, W1, b1, g1, be1, W2, b2, g2, be2, W3, b3, g3, be3):
    xyz = points_data
    xyz_t = jnp.transpose(xyz, (0, 2, 1))
    f0 = jax.random.randint(jax.random.key(42), (B,), 0, N,
                            dtype=jnp.int32).reshape(B, 1)
    centroids_xyz = _fps_centroids(xyz_t, f0)

    d2, thr = _d2_thresholds(centroids_xyz, xyz_t)
    cen_flat = jnp.transpose(centroids_xyz.reshape(_NROWS, 3)).reshape(-1)
    gn_flat = _sc_select_gather(d2.reshape(-1), thr, cen_flat,
                                xyz_t.reshape(-1))
    xrows = jnp.transpose(gn_flat.reshape(3, _ROWS))
    tok = _mlp_tokens(xrows, W1, b1, g1, be1, W2, b2, g2, be2, W3, b3, g3, be3)
    tokens = tok.reshape(B, N_GROUPS, EMBED_DIM)
    return (tokens, centroids_xyz)


# speedup vs baseline: 7.8876x; 1.0863x over previous
import functools

import jax
import jax.numpy as jnp
from jax import lax
from jax.experimental import pallas as pl
from jax.experimental.pallas import tpu as pltpu
from jax.experimental.pallas import tpu_sc as plsc

B = 8
N = 8192
N_GROUPS = 512
GROUP_SIZE = 32
EMBED_DIM = 384


def _fps_body(xyz_ref, f0_ref, cx_ref, cy_ref, cz_ref):
    x = xyz_ref[:, 0, :]
    y = xyz_ref[:, 1, :]
    z = xyz_ref[:, 2, :]
    col = lax.broadcasted_iota(jnp.int32, (B, N), 1)
    colM = lax.broadcasted_iota(jnp.int32, (B, N_GROUPS), 1)

    def body(i, carry):
        dist, far, ax, ay, az = carry
        onehot = (col == far).astype(jnp.float32)
        cx = jnp.sum(x * onehot, axis=1, keepdims=True)
        cy = jnp.sum(y * onehot, axis=1, keepdims=True)
        cz = jnp.sum(z * onehot, axis=1, keepdims=True)
        hit = colM == i
        ax = jnp.where(hit, cx, ax)
        ay = jnp.where(hit, cy, ay)
        az = jnp.where(hit, cz, az)
        dx = x - cx
        dy = y - cy
        dz = z - cz
        d = dx * dx + dy * dy
        d = d + dz * dz
        dist = jnp.where(d < dist, d, dist)
        m = jnp.max(dist, axis=1, keepdims=True)
        sel = jnp.where(dist == m, col, jnp.int32(N))
        far = jnp.min(sel, axis=1, keepdims=True)
        return dist, far, ax, ay, az

    dist0 = jnp.full((B, N), 1e10, dtype=jnp.float32)
    far0 = f0_ref[...]
    zM = jnp.zeros((B, N_GROUPS), dtype=jnp.float32)
    _, _, ax, ay, az = lax.fori_loop(0, N_GROUPS, body,
                                     (dist0, far0, zM, zM, zM))
    cx_ref[...] = ax
    cy_ref[...] = ay
    cz_ref[...] = az


def _fps_centroids(xyz_t, f0):
    cx, cy, cz = pl.pallas_call(
        _fps_body,
        out_shape=[jax.ShapeDtypeStruct((B, N_GROUPS), jnp.float32)] * 3,
    )(xyz_t, f0)
    return jnp.stack([cx, cy, cz], axis=-1)


_ROWS = B * N_GROUPS * GROUP_SIZE


_MBLK = 256
_BISECT = 26


def _d2thr_body(cen_ref, xyz_ref, d2_ref, thr_ref):
    cen = cen_ref[0]
    P = xyz_ref[0]
    G = lax.dot_general(cen, P, (((1,), (0,)), ((), ())),
                        preferred_element_type=jnp.float32)
    cn2 = jnp.sum(P * P, axis=0, keepdims=True)
    cm2 = jnp.sum(cen * cen, axis=1, keepdims=True)
    d2 = (cm2 + cn2) - 2.0 * G
    d2_ref[0] = d2

    cmin = jnp.min(d2.reshape(_MBLK, GROUP_SIZE, N // GROUP_SIZE), axis=2)
    hi0 = jnp.max(cmin, axis=1, keepdims=True)
    lo0 = jnp.min(cmin, axis=1, keepdims=True) - 1.0
    kf = jnp.float32(GROUP_SIZE)

    def bis(_, carry):
        lo, hi, t, found = carry
        mid = 0.5 * (lo + hi)
        cnt = jnp.sum((d2 <= mid).astype(jnp.float32), axis=1, keepdims=True)
        live = found == 0.0
        eq = jnp.logical_and(cnt == kf, live)
        t = jnp.where(eq, mid, t)
        found = jnp.where(eq, 1.0, found)
        ge = cnt >= kf
        live2 = jnp.logical_and(live, jnp.logical_not(eq))
        hi = jnp.where(jnp.logical_and(live2, ge), mid, hi)
        lo = jnp.where(jnp.logical_and(live2, jnp.logical_not(ge)), mid, lo)
        return lo, hi, t, found

    f0 = jnp.zeros((_MBLK, 1), dtype=jnp.float32)
    lo, hi, t, found = lax.fori_loop(0, _BISECT, bis, (lo0, hi0, hi0, f0))
    thr_ref[0] = jnp.where(found > 0.0, t, hi)


def _d2_thresholds(cen, xyz_t):
    d2, thr = pl.pallas_call(
        _d2thr_body,
        grid=(B, N_GROUPS // _MBLK),
        in_specs=[
            pl.BlockSpec((1, _MBLK, 3), lambda b, r: (b, r, 0)),
            pl.BlockSpec((1, 3, N), lambda b, r: (b, 0, 0)),
        ],
        out_specs=[
            pl.BlockSpec((1, _MBLK, N), lambda b, r: (b, r, 0)),
            pl.BlockSpec((1, _MBLK, 1), lambda b, r: (b, r, 0)),
        ],
        out_shape=[
            jax.ShapeDtypeStruct((B, N_GROUPS, N), jnp.float32),
            jax.ShapeDtypeStruct((B, N_GROUPS, 1), jnp.float32),
        ],
        compiler_params=pltpu.CompilerParams(
            dimension_semantics=("arbitrary", "arbitrary")),
    )(cen, xyz_t)
    return d2.reshape(B * N_GROUPS, N), thr.reshape(B * N_GROUPS)


_NW = 32
_NROWS = B * N_GROUPS
_RPT = _NROWS // _NW
_PPT = _RPT * GROUP_SIZE
_NV = N // 16


def _sc_body(d2_hbm, thr_hbm, cen_hbm, pts_hbm, out_hbm,
             d2a, d2b, thr_v, cxv, cyv, czv, xp, yp, zp,
             bufx, bufy, bufz, selbuf, sema, semb):
    cc = lax.axis_index("c")
    ss = lax.axis_index("s")
    wid = ss * 2 + cc
    base_row = wid * _RPT
    bt = wid // (_NW // B)

    pltpu.sync_copy(thr_hbm.at[pl.ds(base_row, _RPT)], thr_v)
    pltpu.sync_copy(cen_hbm.at[pl.ds(base_row, _RPT)], cxv)
    pltpu.sync_copy(cen_hbm.at[pl.ds(_NROWS + base_row, _RPT)], cyv)
    pltpu.sync_copy(cen_hbm.at[pl.ds(2 * _NROWS + base_row, _RPT)], czv)
    pb = bt * 3 * N
    pltpu.sync_copy(pts_hbm.at[pl.ds(pb, N)], xp)
    pltpu.sync_copy(pts_hbm.at[pl.ds(pb + N, N)], yp)
    pltpu.sync_copy(pts_hbm.at[pl.ds(pb + 2 * N, N)], zp)

    iota16 = lax.iota(jnp.int32, 16)

    def process_row(r, d2row):
        rsplat = jnp.broadcast_to(r, (16,)).astype(jnp.int32)
        tv = plsc.load_gather(thr_v, [rsplat])

        def scan_body(ci, p):
            b0 = ci * 128
            vs = [d2row[pl.ds(pl.multiple_of(b0 + k * 16, 16), 16)]
                  for k in range(8)]
            ms = [v <= tv for v in vs]
            pcs = [plsc.all_reduce_population_count(m)[0] for m in ms]
            for k in range(8):
                plsc.store_compressed(selbuf.at[pl.ds(p, 16)],
                                      iota16 + (b0 + k * 16), mask=ms[k])
                p = p + pcs[k]
            return p

        lax.fori_loop(0, _NV // 8, scan_body, jnp.int32(0))

        i0 = selbuf[pl.ds(0, 16)]
        i1 = selbuf[pl.ds(16, 16)]
        cx = plsc.load_gather(cxv, [rsplat])
        cy = plsc.load_gather(cyv, [rsplat])
        cz = plsc.load_gather(czv, [rsplat])
        ob = pl.multiple_of(r * GROUP_SIZE, GROUP_SIZE)
        bufx[pl.ds(ob, 16)] = plsc.load_gather(xp, [i0]) - cx
        bufx[pl.ds(ob + 16, 16)] = plsc.load_gather(xp, [i1]) - cx
        bufy[pl.ds(ob, 16)] = plsc.load_gather(yp, [i0]) - cy
        bufy[pl.ds(ob + 16, 16)] = plsc.load_gather(yp, [i1]) - cy
        bufz[pl.ds(ob, 16)] = plsc.load_gather(zp, [i0]) - cz
        bufz[pl.ds(ob + 16, 16)] = plsc.load_gather(zp, [i1]) - cz

    def rowslice(r):
        return d2_hbm.at[pl.ds(pl.multiple_of((base_row + r) * N, N), N)]

    pltpu.async_copy(rowslice(0), d2a, sema)

    def two_rows(i, _):
        ra = 2 * i
        pltpu.async_copy(rowslice(ra + 1), d2b, semb)
        pltpu.make_async_copy(rowslice(0), d2a, sema).wait()
        process_row(ra, d2a)

        @pl.when(ra + 2 < _RPT)
        def _():
            pltpu.async_copy(rowslice(ra + 2), d2a, sema)

        pltpu.make_async_copy(rowslice(0), d2b, semb).wait()
        process_row(ra + 1, d2b)
        return 0

    lax.fori_loop(0, _RPT // 2, two_rows, 0)

    ob0 = wid * _PPT
    pltpu.sync_copy(bufx, out_hbm.at[pl.ds(ob0, _PPT)])
    pltpu.sync_copy(bufy, out_hbm.at[pl.ds(_ROWS + ob0, _PPT)])
    pltpu.sync_copy(bufz, out_hbm.at[pl.ds(2 * _ROWS + ob0, _PPT)])


_sc_select_gather = functools.partial(
    pl.kernel,
    out_type=jax.ShapeDtypeStruct((3 * _ROWS,), jnp.float32),
    mesh=plsc.VectorSubcoreMesh(core_axis_name="c", subcore_axis_name="s"),
    compiler_params=pltpu.CompilerParams(needs_layout_passes=False),
    scratch_types=[
        pltpu.VMEM((N,), jnp.float32),
        pltpu.VMEM((N,), jnp.float32),
        pltpu.VMEM((_RPT,), jnp.float32),
        pltpu.VMEM((_RPT,), jnp.float32),
        pltpu.VMEM((_RPT,), jnp.float32),
        pltpu.VMEM((_RPT,), jnp.float32),
        pltpu.VMEM((N,), jnp.float32),
        pltpu.VMEM((N,), jnp.float32),
        pltpu.VMEM((N,), jnp.float32),
        pltpu.VMEM((_PPT,), jnp.float32),
        pltpu.VMEM((_PPT,), jnp.float32),
        pltpu.VMEM((_PPT,), jnp.float32),
        pltpu.VMEM((N,), jnp.int32),
        pltpu.SemaphoreType.DMA,
        pltpu.SemaphoreType.DMA,
    ],
)(_sc_body)


_RBLK = 2048
_NBLK = _ROWS // _RBLK
_GBLK = _RBLK // GROUP_SIZE


def _mlp_body(x_ref, W1_ref, b1_ref, g1_ref, be1_ref, W2_ref, b2_ref, g2_ref,
              be2_ref, W3_ref, b3_ref, g3_ref, be3_ref, tok_ref,
              s1, q1, s2, q2, s3, q3, gmax, gmin):
    p = pl.program_id(0)
    j = pl.program_id(1)
    eps = jnp.float32(1e-5)
    ntot = jnp.float32(_ROWS)

    def mm(a, w_ref, b_ref):
        y = lax.dot_general(a, w_ref[...], (((1,), (1,)), ((), ())),
                            preferred_element_type=jnp.float32)
        return y + b_ref[...]

    def bn_relu(y, s_ref, q_ref, g_ref, be_ref):
        m = s_ref[...] / ntot
        var = q_ref[...] / ntot - m * m
        inv = lax.rsqrt(var + eps)
        return jnp.maximum(g_ref[...] * (y - m) * inv + be_ref[...], 0.0)

    @pl.when(jnp.logical_and(p == 0, j == 0))
    def _init():
        s1[...] = jnp.zeros_like(s1)
        q1[...] = jnp.zeros_like(q1)
        s2[...] = jnp.zeros_like(s2)
        q2[...] = jnp.zeros_like(q2)
        s3[...] = jnp.zeros_like(s3)
        q3[...] = jnp.zeros_like(q3)

    @pl.when(p == 0)
    def _p0():
        y1 = mm(x_ref[...], W1_ref, b1_ref)
        s1[...] += jnp.sum(y1, axis=0, keepdims=True)
        q1[...] += jnp.sum(y1 * y1, axis=0, keepdims=True)

    @pl.when(p == 1)
    def _p1():
        y1 = mm(x_ref[...], W1_ref, b1_ref)
        h1 = bn_relu(y1, s1, q1, g1_ref, be1_ref)
        y2 = mm(h1, W2_ref, b2_ref)
        s2[...] += jnp.sum(y2, axis=0, keepdims=True)
        q2[...] += jnp.sum(y2 * y2, axis=0, keepdims=True)

    @pl.when(p == 2)
    def _p2():
        y1 = mm(x_ref[...], W1_ref, b1_ref)
        h1 = bn_relu(y1, s1, q1, g1_ref, be1_ref)
        y2 = mm(h1, W2_ref, b2_ref)
        h2 = bn_relu(y2, s2, q2, g2_ref, be2_ref)
        y3 = mm(h2, W3_ref, b3_ref)
        s3[...] += jnp.sum(y3, axis=0, keepdims=True)
        q3[...] += jnp.sum(y3 * y3, axis=0, keepdims=True)
        y3g = y3.reshape(_GBLK, GROUP_SIZE, EMBED_DIM)
        r0 = pl.multiple_of(j * _GBLK, _GBLK)
        gmax[pl.ds(r0, _GBLK), :] = jnp.max(y3g, axis=1)
        gmin[pl.ds(r0, _GBLK), :] = jnp.min(y3g, axis=1)

    @pl.when(p == 3)
    def _p3():
        m3 = s3[...] / ntot
        var3 = q3[...] / ntot - m3 * m3
        inv3 = lax.rsqrt(var3 + eps)
        r0 = pl.multiple_of(j * _GBLK, _GBLK)
        gx = gmax[pl.ds(r0, _GBLK), :]
        gm = gmin[pl.ds(r0, _GBLK), :]
        g3v = g3_ref[...]
        hi = g3v * (gx - m3) * inv3
        lo = g3v * (gm - m3) * inv3
        tok_ref[...] = jnp.where(g3v > 0, hi, lo) + be3_ref[...]


def _mlp_tokens(xrows, W1, b1, g1, be1, W2, b2, g2, be2, W3, b3, g3, be3):
    r2 = lambda a: a.reshape(1, -1)
    out = pl.pallas_call(
        _mlp_body,
        grid=(4, _NBLK),
        in_specs=[
            pl.BlockSpec((_RBLK, 3), lambda p, j: (j, 0)),
            pl.BlockSpec((64, 3), lambda p, j: (0, 0)),
            pl.BlockSpec((1, 64), lambda p, j: (0, 0)),
            pl.BlockSpec((1, 64), lambda p, j: (0, 0)),
            pl.BlockSpec((1, 64), lambda p, j: (0, 0)),
            pl.BlockSpec((128, 64), lambda p, j: (0, 0)),
            pl.BlockSpec((1, 128), lambda p, j: (0, 0)),
            pl.BlockSpec((1, 128), lambda p, j: (0, 0)),
            pl.BlockSpec((1, 128), lambda p, j: (0, 0)),
            pl.BlockSpec((EMBED_DIM, 128), lambda p, j: (0, 0)),
            pl.BlockSpec((1, EMBED_DIM), lambda p, j: (0, 0)),
            pl.BlockSpec((1, EMBED_DIM), lambda p, j: (0, 0)),
            pl.BlockSpec((1, EMBED_DIM), lambda p, j: (0, 0)),
        ],
        out_specs=pl.BlockSpec((_GBLK, EMBED_DIM), lambda p, j: (j, 0)),
        out_shape=jax.ShapeDtypeStruct((B * N_GROUPS, EMBED_DIM), jnp.float32),
        scratch_shapes=[
            pltpu.VMEM((1, 64), jnp.float32), pltpu.VMEM((1, 64), jnp.float32),
            pltpu.VMEM((1, 128), jnp.float32), pltpu.VMEM((1, 128), jnp.float32),
            pltpu.VMEM((1, EMBED_DIM), jnp.float32),
            pltpu.VMEM((1, EMBED_DIM), jnp.float32),
            pltpu.VMEM((B * N_GROUPS, EMBED_DIM), jnp.float32),
            pltpu.VMEM((B * N_GROUPS, EMBED_DIM), jnp.float32),
        ],
        compiler_params=pltpu.CompilerParams(
            dimension_semantics=("arbitrary", "arbitrary")),
    )(xrows, W1, r2(b1), r2(g1), r2(be1), W2, r2(b2), r2(g2), r2(be2),
      W3, r2(b3), r2(g3), r2(be3))
    return out


def kernel(points_data, W1, b1, g1, be1, W2, b2, g2, be2, W3, b3, g3, be3):
    xyz = points_data
    xyz_t = jnp.transpose(xyz, (0, 2, 1))
    f0 = jax.random.randint(jax.random.key(42), (B,), 0, N,
                            dtype=jnp.int32).reshape(B, 1)
    centroids_xyz = _fps_centroids(xyz_t, f0)

    d2, thr = _d2_thresholds(centroids_xyz, xyz_t)
    cen_flat = jnp.transpose(centroids_xyz.reshape(_NROWS, 3)).reshape(-1)
    gn_flat = _sc_select_gather(d2.reshape(-1), thr, cen_flat,
                                xyz_t.reshape(-1))
    xrows = jnp.transpose(gn_flat.reshape(3, _ROWS))
    tok = _mlp_tokens(xrows, W1, b1, g1, be1, W2, b2, g2, be2, W3, b3, g3, be3)
    tokens = tok.reshape(B, N_GROUPS, EMBED_DIM)
    return (tokens, centroids_xyz)


# speedup vs baseline: 8.1961x; 1.0391x over previous
import functools

import jax
import jax.numpy as jnp
from jax import lax
from jax.experimental import pallas as pl
from jax.experimental.pallas import tpu as pltpu
from jax.experimental.pallas import tpu_sc as plsc

B = 8
N = 8192
N_GROUPS = 512
GROUP_SIZE = 32
EMBED_DIM = 384


def _fps_body(xyz_ref, f0_ref, cx_ref, cy_ref, cz_ref):
    x = xyz_ref[:, 0, :]
    y = xyz_ref[:, 1, :]
    z = xyz_ref[:, 2, :]
    col = lax.broadcasted_iota(jnp.int32, (B, N), 1)
    colM = lax.broadcasted_iota(jnp.int32, (B, N_GROUPS), 1)

    def body(i, carry):
        dist, far, ax, ay, az = carry
        onehot = (col == far).astype(jnp.float32)
        cx = jnp.sum(x * onehot, axis=1, keepdims=True)
        cy = jnp.sum(y * onehot, axis=1, keepdims=True)
        cz = jnp.sum(z * onehot, axis=1, keepdims=True)
        hit = colM == i
        ax = jnp.where(hit, cx, ax)
        ay = jnp.where(hit, cy, ay)
        az = jnp.where(hit, cz, az)
        dx = x - cx
        dy = y - cy
        dz = z - cz
        d = dx * dx + dy * dy
        d = d + dz * dz
        dist = jnp.where(d < dist, d, dist)
        m = jnp.max(dist, axis=1, keepdims=True)
        sel = jnp.where(dist == m, col, jnp.int32(N))
        far = jnp.min(sel, axis=1, keepdims=True)
        return dist, far, ax, ay, az

    dist0 = jnp.full((B, N), 1e10, dtype=jnp.float32)
    far0 = f0_ref[...]
    zM = jnp.zeros((B, N_GROUPS), dtype=jnp.float32)
    _, _, ax, ay, az = lax.fori_loop(0, N_GROUPS, body,
                                     (dist0, far0, zM, zM, zM))
    cx_ref[...] = ax
    cy_ref[...] = ay
    cz_ref[...] = az


def _fps_centroids(xyz_t, f0):
    cx, cy, cz = pl.pallas_call(
        _fps_body,
        out_shape=[jax.ShapeDtypeStruct((B, N_GROUPS), jnp.float32)] * 3,
    )(xyz_t, f0)
    return jnp.stack([cx, cy, cz], axis=-1)


_ROWS = B * N_GROUPS * GROUP_SIZE


_MBLK = 256
_BISECT = 26


def _d2thr_body(cen_ref, xyz_ref, d2_ref, thr_ref):
    cen = cen_ref[0]
    P = xyz_ref[0]
    G = lax.dot_general(cen, P, (((1,), (0,)), ((), ())),
                        preferred_element_type=jnp.float32)
    cn2 = jnp.sum(P * P, axis=0, keepdims=True)
    cm2 = jnp.sum(cen * cen, axis=1, keepdims=True)
    d2 = (cm2 + cn2) - 2.0 * G
    d2_ref[0] = d2

    cmin = jnp.min(d2.reshape(_MBLK, GROUP_SIZE, N // GROUP_SIZE), axis=2)
    hi0 = jnp.max(cmin, axis=1, keepdims=True)
    lo0 = jnp.min(cmin, axis=1, keepdims=True) - 1.0
    kf = jnp.float32(GROUP_SIZE)

    def bis(_, carry):
        lo, hi, t, found = carry
        mid = 0.5 * (lo + hi)
        cnt = jnp.sum((d2 <= mid).astype(jnp.float32), axis=1, keepdims=True)
        live = found == 0.0
        eq = jnp.logical_and(cnt == kf, live)
        t = jnp.where(eq, mid, t)
        found = jnp.where(eq, 1.0, found)
        ge = cnt >= kf
        live2 = jnp.logical_and(live, jnp.logical_not(eq))
        hi = jnp.where(jnp.logical_and(live2, ge), mid, hi)
        lo = jnp.where(jnp.logical_and(live2, jnp.logical_not(ge)), mid, lo)
        return lo, hi, t, found

    f0 = jnp.zeros((_MBLK, 1), dtype=jnp.float32)
    lo, hi, t, found = lax.fori_loop(0, _BISECT, bis, (lo0, hi0, hi0, f0))
    thr_ref[0] = jnp.where(found > 0.0, t, hi)


def _d2_thresholds(cen, xyz_t):
    d2, thr = pl.pallas_call(
        _d2thr_body,
        grid=(B, N_GROUPS // _MBLK),
        in_specs=[
            pl.BlockSpec((1, _MBLK, 3), lambda b, r: (b, r, 0)),
            pl.BlockSpec((1, 3, N), lambda b, r: (b, 0, 0)),
        ],
        out_specs=[
            pl.BlockSpec((1, _MBLK, N), lambda b, r: (b, r, 0)),
            pl.BlockSpec((1, _MBLK, 1), lambda b, r: (b, r, 0)),
        ],
        out_shape=[
            jax.ShapeDtypeStruct((B, N_GROUPS, N), jnp.float32),
            jax.ShapeDtypeStruct((B, N_GROUPS, 1), jnp.float32),
        ],
        compiler_params=pltpu.CompilerParams(
            dimension_semantics=("arbitrary", "arbitrary")),
    )(cen, xyz_t)
    return d2.reshape(B * N_GROUPS, N), thr.reshape(B * N_GROUPS)


_NW = 32
_NROWS = B * N_GROUPS
_RPT = _NROWS // _NW
_PPT = _RPT * GROUP_SIZE
_NV = N // 16


def _sc_body(d2_hbm, thr_hbm, cen_hbm, pts_hbm, out_hbm,
             d2a, d2b, thr_v, cxv, cyv, czv, xp, yp, zp,
             bufx, bufy, bufz, selbuf, sema, semb):
    cc = lax.axis_index("c")
    ss = lax.axis_index("s")
    wid = ss * 2 + cc
    base_row = wid * _RPT
    bt = wid // (_NW // B)

    pltpu.sync_copy(thr_hbm.at[pl.ds(base_row, _RPT)], thr_v)
    pltpu.sync_copy(cen_hbm.at[pl.ds(base_row, _RPT)], cxv)
    pltpu.sync_copy(cen_hbm.at[pl.ds(_NROWS + base_row, _RPT)], cyv)
    pltpu.sync_copy(cen_hbm.at[pl.ds(2 * _NROWS + base_row, _RPT)], czv)
    pb = bt * 3 * N
    pltpu.sync_copy(pts_hbm.at[pl.ds(pb, N)], xp)
    pltpu.sync_copy(pts_hbm.at[pl.ds(pb + N, N)], yp)
    pltpu.sync_copy(pts_hbm.at[pl.ds(pb + 2 * N, N)], zp)

    iota16 = lax.iota(jnp.int32, 16)

    def process_row(r, d2row):
        rsplat = jnp.broadcast_to(r, (16,)).astype(jnp.int32)
        tv = plsc.load_gather(thr_v, [rsplat])

        def scan_body(ci, p):
            b0 = ci * 128
            vs = [d2row[pl.ds(pl.multiple_of(b0 + k * 16, 16), 16)]
                  for k in range(8)]
            ms = [v <= tv for v in vs]
            pcs = [plsc.all_reduce_population_count(m)[0] for m in ms]
            for k in range(8):
                plsc.store_compressed(selbuf.at[pl.ds(p, 16)],
                                      iota16 + (b0 + k * 16), mask=ms[k])
                p = p + pcs[k]
            return p

        lax.fori_loop(0, _NV // 8, scan_body, jnp.int32(0))

        i0 = selbuf[pl.ds(0, 16)]
        i1 = selbuf[pl.ds(16, 16)]
        cx = plsc.load_gather(cxv, [rsplat])
        cy = plsc.load_gather(cyv, [rsplat])
        cz = plsc.load_gather(czv, [rsplat])
        ob = pl.multiple_of(r * GROUP_SIZE, GROUP_SIZE)
        bufx[pl.ds(ob, 16)] = plsc.load_gather(xp, [i0]) - cx
        bufx[pl.ds(ob + 16, 16)] = plsc.load_gather(xp, [i1]) - cx
        bufy[pl.ds(ob, 16)] = plsc.load_gather(yp, [i0]) - cy
        bufy[pl.ds(ob + 16, 16)] = plsc.load_gather(yp, [i1]) - cy
        bufz[pl.ds(ob, 16)] = plsc.load_gather(zp, [i0]) - cz
        bufz[pl.ds(ob + 16, 16)] = plsc.load_gather(zp, [i1]) - cz

    def rowslice(r):
        return d2_hbm.at[pl.ds(pl.multiple_of((base_row + r) * N, N), N)]

    pltpu.async_copy(rowslice(0), d2a, sema)

    def two_rows(i, _):
        ra = 2 * i
        pltpu.async_copy(rowslice(ra + 1), d2b, semb)
        pltpu.make_async_copy(rowslice(0), d2a, sema).wait()
        process_row(ra, d2a)

        @pl.when(ra + 2 < _RPT)
        def _():
            pltpu.async_copy(rowslice(ra + 2), d2a, sema)

        pltpu.make_async_copy(rowslice(0), d2b, semb).wait()
        process_row(ra + 1, d2b)
        return 0

    lax.fori_loop(0, _RPT // 2, two_rows, 0)

    ob0 = wid * _PPT
    pltpu.sync_copy(bufx, out_hbm.at[pl.ds(ob0, _PPT)])
    pltpu.sync_copy(bufy, out_hbm.at[pl.ds(_ROWS + ob0, _PPT)])
    pltpu.sync_copy(bufz, out_hbm.at[pl.ds(2 * _ROWS + ob0, _PPT)])


_sc_select_gather = functools.partial(
    pl.kernel,
    out_type=jax.ShapeDtypeStruct((3 * _ROWS,), jnp.float32),
    mesh=plsc.VectorSubcoreMesh(core_axis_name="c", subcore_axis_name="s"),
    compiler_params=pltpu.CompilerParams(needs_layout_passes=False),
    scratch_types=[
        pltpu.VMEM((N,), jnp.float32),
        pltpu.VMEM((N,), jnp.float32),
        pltpu.VMEM((_RPT,), jnp.float32),
        pltpu.VMEM((_RPT,), jnp.float32),
        pltpu.VMEM((_RPT,), jnp.float32),
        pltpu.VMEM((_RPT,), jnp.float32),
        pltpu.VMEM((N,), jnp.float32),
        pltpu.VMEM((N,), jnp.float32),
        pltpu.VMEM((N,), jnp.float32),
        pltpu.VMEM((_PPT,), jnp.float32),
        pltpu.VMEM((_PPT,), jnp.float32),
        pltpu.VMEM((_PPT,), jnp.float32),
        pltpu.VMEM((N,), jnp.int32),
        pltpu.SemaphoreType.DMA,
        pltpu.SemaphoreType.DMA,
    ],
)(_sc_body)


_RBLK = 2048
_NBLK = _ROWS // _RBLK
_GBLK = _RBLK // GROUP_SIZE


def _mlp_body(x_ref, W1_ref, b1_ref, g1_ref, be1_ref, W2_ref, b2_ref, g2_ref,
              be2_ref, W3_ref, b3_ref, g3_ref, be3_ref, tok_ref,
              s1, q1, s2, q2, s3, q3, gmax, gmin):
    p = pl.program_id(0)
    j = pl.program_id(1)
    eps = jnp.float32(1e-5)
    ntot = jnp.float32(_ROWS)

    def mm(a, w_ref, b_ref):
        y = lax.dot_general(a, w_ref[...], (((1,), (1,)), ((), ())),
                            preferred_element_type=jnp.float32)
        return y + b_ref[...]

    def y1_of():
        y = lax.dot_general(x_ref[...], W1_ref[...], (((0,), (1,)), ((), ())),
                            preferred_element_type=jnp.float32)
        return y + b1_ref[...]

    def bn_relu(y, s_ref, q_ref, g_ref, be_ref):
        m = s_ref[...] / ntot
        var = q_ref[...] / ntot - m * m
        inv = lax.rsqrt(var + eps)
        return jnp.maximum(g_ref[...] * (y - m) * inv + be_ref[...], 0.0)

    @pl.when(jnp.logical_and(p == 0, j == 0))
    def _init():
        s1[...] = jnp.zeros_like(s1)
        q1[...] = jnp.zeros_like(q1)
        s2[...] = jnp.zeros_like(s2)
        q2[...] = jnp.zeros_like(q2)
        s3[...] = jnp.zeros_like(s3)
        q3[...] = jnp.zeros_like(q3)

    @pl.when(p == 0)
    def _p0():
        y1 = y1_of()
        s1[...] += jnp.sum(y1, axis=0, keepdims=True)
        q1[...] += jnp.sum(y1 * y1, axis=0, keepdims=True)

    @pl.when(p == 1)
    def _p1():
        y1 = y1_of()
        h1 = bn_relu(y1, s1, q1, g1_ref, be1_ref)
        y2 = mm(h1, W2_ref, b2_ref)
        s2[...] += jnp.sum(y2, axis=0, keepdims=True)
        q2[...] += jnp.sum(y2 * y2, axis=0, keepdims=True)

    @pl.when(p == 2)
    def _p2():
        y1 = y1_of()
        h1 = bn_relu(y1, s1, q1, g1_ref, be1_ref)
        y2 = mm(h1, W2_ref, b2_ref)
        h2 = bn_relu(y2, s2, q2, g2_ref, be2_ref)
        y3 = mm(h2, W3_ref, b3_ref)
        s3[...] += jnp.sum(y3, axis=0, keepdims=True)
        q3[...] += jnp.sum(y3 * y3, axis=0, keepdims=True)
        y3g = y3.reshape(_GBLK, GROUP_SIZE, EMBED_DIM)
        r0 = pl.multiple_of(j * _GBLK, _GBLK)
        gmax[pl.ds(r0, _GBLK), :] = jnp.max(y3g, axis=1)
        gmin[pl.ds(r0, _GBLK), :] = jnp.min(y3g, axis=1)

    @pl.when(p == 3)
    def _p3():
        m3 = s3[...] / ntot
        var3 = q3[...] / ntot - m3 * m3
        inv3 = lax.rsqrt(var3 + eps)
        r0 = pl.multiple_of(j * _GBLK, _GBLK)
        gx = gmax[pl.ds(r0, _GBLK), :]
        gm = gmin[pl.ds(r0, _GBLK), :]
        g3v = g3_ref[...]
        hi = g3v * (gx - m3) * inv3
        lo = g3v * (gm - m3) * inv3
        tok_ref[...] = jnp.where(g3v > 0, hi, lo) + be3_ref[...]


def _mlp_tokens(xrows, W1, b1, g1, be1, W2, b2, g2, be2, W3, b3, g3, be3):
    r2 = lambda a: a.reshape(1, -1)
    out = pl.pallas_call(
        _mlp_body,
        grid=(4, _NBLK),
        in_specs=[
            pl.BlockSpec((3, _RBLK), lambda p, j: (0, j)),
            pl.BlockSpec((64, 3), lambda p, j: (0, 0)),
            pl.BlockSpec((1, 64), lambda p, j: (0, 0)),
            pl.BlockSpec((1, 64), lambda p, j: (0, 0)),
            pl.BlockSpec((1, 64), lambda p, j: (0, 0)),
            pl.BlockSpec((128, 64), lambda p, j: (0, 0)),
            pl.BlockSpec((1, 128), lambda p, j: (0, 0)),
            pl.BlockSpec((1, 128), lambda p, j: (0, 0)),
            pl.BlockSpec((1, 128), lambda p, j: (0, 0)),
            pl.BlockSpec((EMBED_DIM, 128), lambda p, j: (0, 0)),
            pl.BlockSpec((1, EMBED_DIM), lambda p, j: (0, 0)),
            pl.BlockSpec((1, EMBED_DIM), lambda p, j: (0, 0)),
            pl.BlockSpec((1, EMBED_DIM), lambda p, j: (0, 0)),
        ],
        out_specs=pl.BlockSpec((_GBLK, EMBED_DIM), lambda p, j: (j, 0)),
        out_shape=jax.ShapeDtypeStruct((B * N_GROUPS, EMBED_DIM), jnp.float32),
        scratch_shapes=[
            pltpu.VMEM((1, 64), jnp.float32), pltpu.VMEM((1, 64), jnp.float32),
            pltpu.VMEM((1, 128), jnp.float32), pltpu.VMEM((1, 128), jnp.float32),
            pltpu.VMEM((1, EMBED_DIM), jnp.float32),
            pltpu.VMEM((1, EMBED_DIM), jnp.float32),
            pltpu.VMEM((B * N_GROUPS, EMBED_DIM), jnp.float32),
            pltpu.VMEM((B * N_GROUPS, EMBED_DIM), jnp.float32),
        ],
        compiler_params=pltpu.CompilerParams(
            dimension_semantics=("arbitrary", "arbitrary")),
    )(xrows, W1, r2(b1), r2(g1), r2(be1), W2, r2(b2), r2(g2), r2(be2),
      W3, r2(b3), r2(g3), r2(be3))
    return out


def kernel(points_data, W1, b1, g1, be1, W2, b2, g2, be2, W3, b3, g3, be3):
    xyz = points_data
    xyz_t = jnp.transpose(xyz, (0, 2, 1))
    f0 = jax.random.randint(jax.random.key(42), (B,), 0, N,
                            dtype=jnp.int32).reshape(B, 1)
    centroids_xyz = _fps_centroids(xyz_t, f0)

    d2, thr = _d2_thresholds(centroids_xyz, xyz_t)
    cen_flat = jnp.transpose(centroids_xyz.reshape(_NROWS, 3)).reshape(-1)
    gn_flat = _sc_select_gather(d2.reshape(-1), thr, cen_flat,
                                xyz_t.reshape(-1))
    xrows = gn_flat.reshape(3, _ROWS)
    tok = _mlp_tokens(xrows, W1, b1, g1, be1, W2, b2, g2, be2, W3, b3, g3, be3)
    tokens = tok.reshape(B, N_GROUPS, EMBED_DIM)
    return (tokens, centroids_xyz)


# speedup vs baseline: 8.6213x; 1.0519x over previous
import functools

import jax
import jax.numpy as jnp
from jax import lax
from jax.experimental import pallas as pl
from jax.experimental.pallas import tpu as pltpu
from jax.experimental.pallas import tpu_sc as plsc

B = 8
N = 8192
N_GROUPS = 512
GROUP_SIZE = 32
EMBED_DIM = 384


def _fps_body(xyz_ref, f0_ref, cx_ref, cy_ref, cz_ref):
    x = xyz_ref[:, 0, :]
    y = xyz_ref[:, 1, :]
    z = xyz_ref[:, 2, :]
    col = lax.broadcasted_iota(jnp.int32, (B, N), 1)
    colM = lax.broadcasted_iota(jnp.int32, (B, N_GROUPS), 1)

    def body(i, carry):
        dist, far, ax, ay, az = carry
        onehot = (col == far).astype(jnp.float32)
        cx = jnp.sum(x * onehot, axis=1, keepdims=True)
        cy = jnp.sum(y * onehot, axis=1, keepdims=True)
        cz = jnp.sum(z * onehot, axis=1, keepdims=True)
        hit = colM == i
        ax = jnp.where(hit, cx, ax)
        ay = jnp.where(hit, cy, ay)
        az = jnp.where(hit, cz, az)
        dx = x - cx
        dy = y - cy
        dz = z - cz
        d = dx * dx + dy * dy
        d = d + dz * dz
        dist = jnp.where(d < dist, d, dist)
        m = jnp.max(dist, axis=1, keepdims=True)
        sel = jnp.where(dist == m, col, jnp.int32(N))
        far = jnp.min(sel, axis=1, keepdims=True)
        return dist, far, ax, ay, az

    dist0 = jnp.full((B, N), 1e10, dtype=jnp.float32)
    far0 = f0_ref[...]
    zM = jnp.zeros((B, N_GROUPS), dtype=jnp.float32)
    _, _, ax, ay, az = lax.fori_loop(0, N_GROUPS, body,
                                     (dist0, far0, zM, zM, zM))
    cx_ref[...] = ax
    cy_ref[...] = ay
    cz_ref[...] = az


def _fps_centroids(xyz_t, f0):
    cx, cy, cz = pl.pallas_call(
        _fps_body,
        out_shape=[jax.ShapeDtypeStruct((B, N_GROUPS), jnp.float32)] * 3,
    )(xyz_t, f0)
    return jnp.stack([cx, cy, cz], axis=-1)


_ROWS = B * N_GROUPS * GROUP_SIZE


_MBLK = 256
_BISECT = 26


def _d2thr_body(cen_ref, xyz_ref, d2_ref, thr_ref):
    cen = cen_ref[0]
    P = xyz_ref[0]
    G = lax.dot_general(cen, P, (((1,), (0,)), ((), ())),
                        preferred_element_type=jnp.float32)
    cn2 = jnp.sum(P * P, axis=0, keepdims=True)
    cm2 = jnp.sum(cen * cen, axis=1, keepdims=True)
    d2 = (cm2 + cn2) - 2.0 * G
    d2_ref[0] = d2.reshape(_MBLK, N // 128, 128)

    cmin = jnp.min(d2.reshape(_MBLK, GROUP_SIZE, N // GROUP_SIZE), axis=2)
    hi0 = jnp.max(cmin, axis=1, keepdims=True)
    lo0 = jnp.min(cmin, axis=1, keepdims=True) - 1.0
    kf = jnp.float32(GROUP_SIZE)

    def bis(_, carry):
        lo, hi, t, found = carry
        mid = 0.5 * (lo + hi)
        cnt = jnp.sum((d2 <= mid).astype(jnp.float32), axis=1, keepdims=True)
        live = found == 0.0
        eq = jnp.logical_and(cnt == kf, live)
        t = jnp.where(eq, mid, t)
        found = jnp.where(eq, 1.0, found)
        ge = cnt >= kf
        live2 = jnp.logical_and(live, jnp.logical_not(eq))
        hi = jnp.where(jnp.logical_and(live2, ge), mid, hi)
        lo = jnp.where(jnp.logical_and(live2, jnp.logical_not(ge)), mid, lo)
        return lo, hi, t, found

    f0 = jnp.zeros((_MBLK, 1), dtype=jnp.float32)
    lo, hi, t, found = lax.fori_loop(0, _BISECT, bis, (lo0, hi0, hi0, f0))
    thr_ref[0] = jnp.where(found > 0.0, t, hi)


def _d2_thresholds(cen, xyz_t):
    d2, thr = pl.pallas_call(
        _d2thr_body,
        grid=(B, N_GROUPS // _MBLK),
        in_specs=[
            pl.BlockSpec((1, _MBLK, 3), lambda b, r: (b, r, 0)),
            pl.BlockSpec((1, 3, N), lambda b, r: (b, 0, 0)),
        ],
        out_specs=[
            pl.BlockSpec((1, _MBLK, N // 128, 128), lambda b, r: (b, r, 0, 0)),
            pl.BlockSpec((1, _MBLK, 1), lambda b, r: (b, r, 0)),
        ],
        out_shape=[
            jax.ShapeDtypeStruct((B, N_GROUPS, N // 128, 128), jnp.float32),
            jax.ShapeDtypeStruct((B, N_GROUPS, 1), jnp.float32),
        ],
        compiler_params=pltpu.CompilerParams(
            dimension_semantics=("arbitrary", "arbitrary")),
    )(cen, xyz_t)
    return d2.reshape(B * N_GROUPS, N), thr.reshape(B * N_GROUPS)


_NW = 32
_NROWS = B * N_GROUPS
_RPT = _NROWS // _NW
_PPT = _RPT * GROUP_SIZE
_NV = N // 16


def _sc_body(d2_hbm, thr_hbm, cen_hbm, pts_hbm, out_hbm,
             d2a, d2b, thr_v, cxv, cyv, czv, xp, yp, zp,
             bufx, bufy, bufz, selbuf, sema, semb):
    cc = lax.axis_index("c")
    ss = lax.axis_index("s")
    wid = ss * 2 + cc
    base_row = wid * _RPT
    bt = wid // (_NW // B)

    pltpu.sync_copy(thr_hbm.at[pl.ds(base_row, _RPT)], thr_v)
    pltpu.sync_copy(cen_hbm.at[pl.ds(base_row, _RPT)], cxv)
    pltpu.sync_copy(cen_hbm.at[pl.ds(_NROWS + base_row, _RPT)], cyv)
    pltpu.sync_copy(cen_hbm.at[pl.ds(2 * _NROWS + base_row, _RPT)], czv)
    pb = bt * 3 * N
    pltpu.sync_copy(pts_hbm.at[pl.ds(pb, N)], xp)
    pltpu.sync_copy(pts_hbm.at[pl.ds(pb + N, N)], yp)
    pltpu.sync_copy(pts_hbm.at[pl.ds(pb + 2 * N, N)], zp)

    iota16 = lax.iota(jnp.int32, 16)

    def process_row(r, d2row):
        rsplat = jnp.broadcast_to(r, (16,)).astype(jnp.int32)
        tv = plsc.load_gather(thr_v, [rsplat])

        def scan_body(ci, p):
            b0 = ci * 128
            vs = [d2row[pl.ds(pl.multiple_of(b0 + k * 16, 16), 16)]
                  for k in range(8)]
            ms = [v <= tv for v in vs]
            pcs = [plsc.all_reduce_population_count(m)[0] for m in ms]
            for k in range(8):
                plsc.store_compressed(selbuf.at[pl.ds(p, 16)],
                                      iota16 + (b0 + k * 16), mask=ms[k])
                p = p + pcs[k]
            return p

        lax.fori_loop(0, _NV // 8, scan_body, jnp.int32(0))

        i0 = selbuf[pl.ds(0, 16)]
        i1 = selbuf[pl.ds(16, 16)]
        cx = plsc.load_gather(cxv, [rsplat])
        cy = plsc.load_gather(cyv, [rsplat])
        cz = plsc.load_gather(czv, [rsplat])
        ob = pl.multiple_of(r * GROUP_SIZE, GROUP_SIZE)
        bufx[pl.ds(ob, 16)] = plsc.load_gather(xp, [i0]) - cx
        bufx[pl.ds(ob + 16, 16)] = plsc.load_gather(xp, [i1]) - cx
        bufy[pl.ds(ob, 16)] = plsc.load_gather(yp, [i0]) - cy
        bufy[pl.ds(ob + 16, 16)] = plsc.load_gather(yp, [i1]) - cy
        bufz[pl.ds(ob, 16)] = plsc.load_gather(zp, [i0]) - cz
        bufz[pl.ds(ob + 16, 16)] = plsc.load_gather(zp, [i1]) - cz

    def rowslice(r):
        return d2_hbm.at[pl.ds(pl.multiple_of((base_row + r) * N, N), N)]

    pltpu.async_copy(rowslice(0), d2a, sema)

    def two_rows(i, _):
        ra = 2 * i
        pltpu.async_copy(rowslice(ra + 1), d2b, semb)
        pltpu.make_async_copy(rowslice(0), d2a, sema).wait()
        process_row(ra, d2a)

        @pl.when(ra + 2 < _RPT)
        def _():
            pltpu.async_copy(rowslice(ra + 2), d2a, sema)

        pltpu.make_async_copy(rowslice(0), d2b, semb).wait()
        process_row(ra + 1, d2b)
        return 0

    lax.fori_loop(0, _RPT // 2, two_rows, 0)

    ob0 = wid * _PPT
    pltpu.sync_copy(bufx, out_hbm.at[pl.ds(ob0, _PPT)])
    pltpu.sync_copy(bufy, out_hbm.at[pl.ds(_ROWS + ob0, _PPT)])
    pltpu.sync_copy(bufz, out_hbm.at[pl.ds(2 * _ROWS + ob0, _PPT)])


_sc_select_gather = functools.partial(
    pl.kernel,
    out_type=jax.ShapeDtypeStruct((3 * _ROWS,), jnp.float32),
    mesh=plsc.VectorSubcoreMesh(core_axis_name="c", subcore_axis_name="s"),
    compiler_params=pltpu.CompilerParams(needs_layout_passes=False),
    scratch_types=[
        pltpu.VMEM((N,), jnp.float32),
        pltpu.VMEM((N,), jnp.float32),
        pltpu.VMEM((_RPT,), jnp.float32),
        pltpu.VMEM((_RPT,), jnp.float32),
        pltpu.VMEM((_RPT,), jnp.float32),
        pltpu.VMEM((_RPT,), jnp.float32),
        pltpu.VMEM((N,), jnp.float32),
        pltpu.VMEM((N,), jnp.float32),
        pltpu.VMEM((N,), jnp.float32),
        pltpu.VMEM((_PPT,), jnp.float32),
        pltpu.VMEM((_PPT,), jnp.float32),
        pltpu.VMEM((_PPT,), jnp.float32),
        pltpu.VMEM((N,), jnp.int32),
        pltpu.SemaphoreType.DMA,
        pltpu.SemaphoreType.DMA,
    ],
)(_sc_body)


_RBLK = 2048
_NBLK = _ROWS // _RBLK
_GBLK = _RBLK // GROUP_SIZE


def _mlp_body(x_ref, W1_ref, b1_ref, g1_ref, be1_ref, W2_ref, b2_ref, g2_ref,
              be2_ref, W3_ref, b3_ref, g3_ref, be3_ref, tok_ref,
              s1, q1, s2, q2, s3, q3, gmax, gmin):
    p = pl.program_id(0)
    j = pl.program_id(1)
    eps = jnp.float32(1e-5)
    ntot = jnp.float32(_ROWS)

    def mm(a, w_ref, b_ref):
        y = lax.dot_general(a, w_ref[...], (((1,), (1,)), ((), ())),
                            preferred_element_type=jnp.float32)
        return y + b_ref[...]

    def y1_of():
        y = lax.dot_general(x_ref[...], W1_ref[...], (((0,), (1,)), ((), ())),
                            preferred_element_type=jnp.float32)
        return y + b1_ref[...]

    def bn_relu(y, s_ref, q_ref, g_ref, be_ref):
        m = s_ref[...] / ntot
        var = q_ref[...] / ntot - m * m
        inv = lax.rsqrt(var + eps)
        return jnp.maximum(g_ref[...] * (y - m) * inv + be_ref[...], 0.0)

    @pl.when(jnp.logical_and(p == 0, j == 0))
    def _init():
        s1[...] = jnp.zeros_like(s1)
        q1[...] = jnp.zeros_like(q1)
        s2[...] = jnp.zeros_like(s2)
        q2[...] = jnp.zeros_like(q2)
        s3[...] = jnp.zeros_like(s3)
        q3[...] = jnp.zeros_like(q3)

    @pl.when(p == 0)
    def _p0():
        y1 = y1_of()
        s1[...] += jnp.sum(y1, axis=0, keepdims=True)
        q1[...] += jnp.sum(y1 * y1, axis=0, keepdims=True)

    @pl.when(p == 1)
    def _p1():
        y1 = y1_of()
        h1 = bn_relu(y1, s1, q1, g1_ref, be1_ref)
        y2 = mm(h1, W2_ref, b2_ref)
        s2[...] += jnp.sum(y2, axis=0, keepdims=True)
        q2[...] += jnp.sum(y2 * y2, axis=0, keepdims=True)

    @pl.when(p == 2)
    def _p2():
        y1 = y1_of()
        h1 = bn_relu(y1, s1, q1, g1_ref, be1_ref)
        y2 = mm(h1, W2_ref, b2_ref)
        h2 = bn_relu(y2, s2, q2, g2_ref, be2_ref)
        y3 = mm(h2, W3_ref, b3_ref)
        s3[...] += jnp.sum(y3, axis=0, keepdims=True)
        q3[...] += jnp.sum(y3 * y3, axis=0, keepdims=True)
        y3g = y3.reshape(_GBLK, GROUP_SIZE, EMBED_DIM)
        r0 = pl.multiple_of(j * _GBLK, _GBLK)
        gmax[pl.ds(r0, _GBLK), :] = jnp.max(y3g, axis=1)
        gmin[pl.ds(r0, _GBLK), :] = jnp.min(y3g, axis=1)

    @pl.when(p == 3)
    def _p3():
        m3 = s3[...] / ntot
        var3 = q3[...] / ntot - m3 * m3
        inv3 = lax.rsqrt(var3 + eps)
        r0 = pl.multiple_of(j * _GBLK, _GBLK)
        gx = gmax[pl.ds(r0, _GBLK), :]
        gm = gmin[pl.ds(r0, _GBLK), :]
        g3v = g3_ref[...]
        hi = g3v * (gx - m3) * inv3
        lo = g3v * (gm - m3) * inv3
        tok_ref[...] = jnp.where(g3v > 0, hi, lo) + be3_ref[...]


def _mlp_tokens(xrows, W1, b1, g1, be1, W2, b2, g2, be2, W3, b3, g3, be3):
    r2 = lambda a: a.reshape(1, -1)
    out = pl.pallas_call(
        _mlp_body,
        grid=(4, _NBLK),
        in_specs=[
            pl.BlockSpec((3, _RBLK), lambda p, j: (0, j)),
            pl.BlockSpec((64, 3), lambda p, j: (0, 0)),
            pl.BlockSpec((1, 64), lambda p, j: (0, 0)),
            pl.BlockSpec((1, 64), lambda p, j: (0, 0)),
            pl.BlockSpec((1, 64), lambda p, j: (0, 0)),
            pl.BlockSpec((128, 64), lambda p, j: (0, 0)),
            pl.BlockSpec((1, 128), lambda p, j: (0, 0)),
            pl.BlockSpec((1, 128), lambda p, j: (0, 0)),
            pl.BlockSpec((1, 128), lambda p, j: (0, 0)),
            pl.BlockSpec((EMBED_DIM, 128), lambda p, j: (0, 0)),
            pl.BlockSpec((1, EMBED_DIM), lambda p, j: (0, 0)),
            pl.BlockSpec((1, EMBED_DIM), lambda p, j: (0, 0)),
            pl.BlockSpec((1, EMBED_DIM), lambda p, j: (0, 0)),
        ],
        out_specs=pl.BlockSpec((_GBLK, EMBED_DIM), lambda p, j: (j, 0)),
        out_shape=jax.ShapeDtypeStruct((B * N_GROUPS, EMBED_DIM), jnp.float32),
        scratch_shapes=[
            pltpu.VMEM((1, 64), jnp.float32), pltpu.VMEM((1, 64), jnp.float32),
            pltpu.VMEM((1, 128), jnp.float32), pltpu.VMEM((1, 128), jnp.float32),
            pltpu.VMEM((1, EMBED_DIM), jnp.float32),
            pltpu.VMEM((1, EMBED_DIM), jnp.float32),
            pltpu.VMEM((B * N_GROUPS, EMBED_DIM), jnp.float32),
            pltpu.VMEM((B * N_GROUPS, EMBED_DIM), jnp.float32),
        ],
        compiler_params=pltpu.CompilerParams(
            dimension_semantics=("arbitrary", "arbitrary")),
    )(xrows, W1, r2(b1), r2(g1), r2(be1), W2, r2(b2), r2(g2), r2(be2),
      W3, r2(b3), r2(g3), r2(be3))
    return out


def kernel(points_data, W1, b1, g1, be1, W2, b2, g2, be2, W3, b3, g3, be3):
    xyz = points_data
    xyz_t = jnp.transpose(xyz, (0, 2, 1))
    f0 = jax.random.randint(jax.random.key(42), (B,), 0, N,
                            dtype=jnp.int32).reshape(B, 1)
    centroids_xyz = _fps_centroids(xyz_t, f0)

    d2, thr = _d2_thresholds(centroids_xyz, xyz_t)
    cen_flat = jnp.transpose(centroids_xyz.reshape(_NROWS, 3)).reshape(-1)
    gn_flat = _sc_select_gather(d2.reshape(-1), thr, cen_flat,
                                xyz_t.reshape(-1))
    xrows = gn_flat.reshape(3, _ROWS)
    tok = _mlp_tokens(xrows, W1, b1, g1, be1, W2, b2, g2, be2, W3, b3, g3, be3)
    tokens = tok.reshape(B, N_GROUPS, EMBED_DIM)
    return (tokens, centroids_xyz)
